# baseline (speedup 1.0000x reference)
import functools

import jax
import jax.numpy as jnp
from jax import lax
from jax.experimental import pallas as pl
from jax.experimental.pallas import tpu as pltpu
from jax.experimental.pallas import tpu_sc as plsc

_B = 16384
_KD = 128
_TD = 64
_NC = 2
_NS = 16
_NW = _NC * _NS
_PER_W = _B // _NW
_C = 128
_ROUNDS = _PER_W // _C
_GROUPS = _C // 16
_NSTAT = 13
_REG = 1e-05


def _sc_body(u_ref, pos_ref, neg_ref, ua_ref, ea_ref, ut_ref, it_ref,
             wu_ref, wc_ref, out_ref,
             idx_u, idx_p, idx_n, ru, rp, rn, tu, tp, tn,
             wus, wps, wns, staging, sem):
  wid = lax.axis_index("s") * _NC + lax.axis_index("c")
  base = wid * _PER_W
  lane = lax.iota(jnp.int32, 16)
  zeros16 = jnp.zeros((16,), jnp.int32)

  for r in range(_ROUNDS):
    off = base + r * _C
    pltpu.sync_copy(u_ref.at[pl.ds(off, _C)], idx_u)
    pltpu.sync_copy(pos_ref.at[pl.ds(off, _C)], idx_p)
    pltpu.sync_copy(neg_ref.at[pl.ds(off, _C)], idx_n)
    cps = [
        pltpu.async_copy(ua_ref.at[idx_u], ru, sem),
        pltpu.async_copy(ea_ref.at[idx_p], rp, sem),
        pltpu.async_copy(ea_ref.at[idx_n], rn, sem),
        pltpu.async_copy(ut_ref.at[idx_u], tu, sem),
        pltpu.async_copy(it_ref.at[idx_p], tp, sem),
        pltpu.async_copy(it_ref.at[idx_n], tn, sem),
        pltpu.async_copy(wu_ref.at[idx_u], wus, sem),
        pltpu.async_copy(wc_ref.at[idx_p], wps, sem),
        pltpu.async_copy(wc_ref.at[idx_n], wns, sem),
    ]
    for cp in cps:
      cp.wait()

    for g in range(_GROUPS):
      rows = lane + (g * 16)
      z = jnp.zeros((16,), jnp.float32)

      def kbody(d, accs, rows=rows):
        aup, aun, auu, app, ann = accs
        col = jnp.full((16,), d, jnp.int32)
        uv = plsc.load_gather(ru, [rows, col])
        pv = plsc.load_gather(rp, [rows, col])
        nv = plsc.load_gather(rn, [rows, col])
        return (aup + uv * pv, aun + uv * nv,
                auu + uv * uv, app + pv * pv, ann + nv * nv)

      aup, aun, auu, app, ann = lax.fori_loop(
          0, _KD, kbody, (z, z, z, z, z), unroll=4)

      def tbody(d, accs, rows=rows):
        bup, bun, buu, bpp, bnn = accs
        col = jnp.full((16,), d, jnp.int32)
        uv = plsc.load_gather(tu, [rows, col])
        pv = plsc.load_gather(tp, [rows, col])
        nv = plsc.load_gather(tn, [rows, col])
        return (bup + uv * pv, bun + uv * nv,
                buu + uv * uv, bpp + pv * pv, bnn + nv * nv)

      bup, bun, buu, bpp, bnn = lax.fori_loop(
          0, _TD, tbody, (z, z, z, z, z), unroll=4)

      wuv = plsc.load_gather(wus, [rows, zeros16])
      wpv = plsc.load_gather(wps, [rows, zeros16])
      wnv = plsc.load_gather(wns, [rows, zeros16])

      o = r * _C + g * 16
      for j, val in enumerate((aup, aun, auu, app, ann,
                               bup, bun, buu, bpp, bnn,
                               wuv, wpv, wnv)):
        staging[pl.ds(j * _PER_W + o, 16)] = val

  pltpu.sync_copy(staging, out_ref.at[wid])


_sc_stats = functools.partial(
    pl.kernel,
    out_type=jax.ShapeDtypeStruct((_NW, _NSTAT * _PER_W), jnp.float32),
    mesh=plsc.VectorSubcoreMesh(
        core_axis_name="c", subcore_axis_name="s",
        num_cores=_NC, num_subcores=_NS),
    scratch_types=[
        pltpu.VMEM((_C,), jnp.int32),
        pltpu.VMEM((_C,), jnp.int32),
        pltpu.VMEM((_C,), jnp.int32),
        pltpu.VMEM((_C, _KD), jnp.float32),
        pltpu.VMEM((_C, _KD), jnp.float32),
        pltpu.VMEM((_C, _KD), jnp.float32),
        pltpu.VMEM((_C, _TD), jnp.float32),
        pltpu.VMEM((_C, _TD), jnp.float32),
        pltpu.VMEM((_C, _TD), jnp.float32),
        pltpu.VMEM((_C, 1), jnp.float32),
        pltpu.VMEM((_C, 1), jnp.float32),
        pltpu.VMEM((_C, 1), jnp.float32),
        pltpu.VMEM((_NSTAT * _PER_W,), jnp.float32),
        pltpu.SemaphoreType.DMA,
    ],
    compiler_params=pltpu.CompilerParams(
        needs_layout_passes=False, use_tc_tiling_on_sc=False),
)(_sc_body)


def _ep_body(s_ref, o_ref):
  x = s_ref[...]

  def st(j):
    return x[:, j * _PER_W:(j + 1) * _PER_W]

  up, un, uu, pp, nn = st(0), st(1), st(2), st(3), st(4)
  tup, tun, tuu, tpp, tnn = st(5), st(6), st(7), st(8), st(9)
  wu, wp, wn = st(10), st(11), st(12)

  eps = jnp.float32(1e-12)
  nu = jnp.maximum(jnp.sqrt(tuu), eps)
  np_ = jnp.maximum(jnp.sqrt(tpp), eps)
  nn_ = jnp.maximum(jnp.sqrt(tnn), eps)

  pos = up + wu * wp * tup / (nu * np_)
  neg = un + wu * wn * tun / (nu * nn_)
  l2 = (uu + pp + nn
        + (wu * wu) * tuu / (nu * nu)
        + (wp * wp) * tpp / (np_ * np_)
        + (wn * wn) * tnn / (nn_ * nn_))

  d = pos - neg
  base = jnp.maximum(-d, 0.0) + jnp.log1p(jnp.exp(-jnp.abs(d)))
  loss = jnp.mean(base) + _REG * (jnp.sum(l2) / (2.0 * _B))
  o_ref[0, 0] = loss


def kernel(u, pos_i, neg_i, ua_embed, ea_embed, u_text, i_text,
           w_utext, w_ctext):
  stats = _sc_stats(u, pos_i, neg_i, ua_embed, ea_embed, u_text, i_text,
                    w_utext, w_ctext)
  loss = pl.pallas_call(
      _ep_body,
      out_shape=jax.ShapeDtypeStruct((1, 1), jnp.float32),
      out_specs=pl.BlockSpec(memory_space=pltpu.SMEM),
  )(stats)
  return loss[0, 0]


# speedup vs baseline: 2.6314x; 2.6314x over previous
import functools

import jax
import jax.numpy as jnp
from jax import lax
from jax.experimental import pallas as pl
from jax.experimental.pallas import tpu as pltpu
from jax.experimental.pallas import tpu_sc as plsc

_B = 16384
_KD = 128
_TD = 64
_NC = 2
_NS = 16
_NW = _NC * _NS
_PER_W = _B // _NW
_C = 128
_ROUNDS = _PER_W // _C
_GROUPS = _C // 16
_REG = 1e-05


def _sc_body(u_ref, pos_ref, neg_ref, ua_ref, ea_ref, ut_ref, it_ref,
             wu_ref, wc_ref,
             o0, o1, o2, o3, o4, o5, o6, o7, o8, o9, o10, o11, o12,
             idx_u, idx_p, idx_n, tix_u, tix_p, tix_n,
             ru, rp, rn, tu, tp, tn, wus, wps, wns, staging, sem):
  outs = (o0, o1, o2, o3, o4, o5, o6, o7, o8, o9, o10, o11, o12)
  wid = lax.axis_index("s") * _NC + lax.axis_index("c")
  base = wid * _PER_W
  lane = lax.iota(jnp.int32, 16)

  for r in range(_ROUNDS):
    off = base + r * _C
    pltpu.sync_copy(u_ref.at[pl.ds(off, _C)], idx_u)
    pltpu.sync_copy(pos_ref.at[pl.ds(off, _C)], idx_p)
    pltpu.sync_copy(neg_ref.at[pl.ds(off, _C)], idx_n)
    for g in range(_GROUPS):
      sl = pl.ds(g * 16, 16)
      tix_u[sl] = lax.shift_right_logical(idx_u[sl], 1)
      tix_p[sl] = lax.shift_right_logical(idx_p[sl], 1)
      tix_n[sl] = lax.shift_right_logical(idx_n[sl], 1)
    cps = [
        pltpu.async_copy(ua_ref.at[idx_u], ru, sem),
        pltpu.async_copy(ea_ref.at[idx_p], rp, sem),
        pltpu.async_copy(ea_ref.at[idx_n], rn, sem),
        pltpu.async_copy(ut_ref.at[tix_u], tu, sem),
        pltpu.async_copy(it_ref.at[tix_p], tp, sem),
        pltpu.async_copy(it_ref.at[tix_n], tn, sem),
        pltpu.async_copy(wu_ref.at[idx_u], wus, sem),
        pltpu.async_copy(wc_ref.at[idx_p], wps, sem),
        pltpu.async_copy(wc_ref.at[idx_n], wns, sem),
    ]
    for cp in cps:
      cp.wait()

    for g in range(_GROUPS):
      rows = lane + (g * 16)
      z = jnp.zeros((16,), jnp.float32)

      def kbody(d, accs, rows=rows):
        aup, aun, auu, app, ann = accs
        col = lax.bitwise_and(lane + d, _KD - 1)
        uv = plsc.load_gather(ru, [rows, col])
        pv = plsc.load_gather(rp, [rows, col])
        nv = plsc.load_gather(rn, [rows, col])
        return (aup + uv * pv, aun + uv * nv,
                auu + uv * uv, app + pv * pv, ann + nv * nv)

      aup, aun, auu, app, ann = lax.fori_loop(
          0, _KD, kbody, (z, z, z, z, z), unroll=4)

      half_u = lax.bitwise_and(idx_u[pl.ds(g * 16, 16)], 1) * _TD
      half_p = lax.bitwise_and(idx_p[pl.ds(g * 16, 16)], 1) * _TD
      half_n = lax.bitwise_and(idx_n[pl.ds(g * 16, 16)], 1) * _TD

      def tbody(d, accs, rows=rows, hu=half_u, hp=half_p, hn=half_n):
        bup, bun, buu, bpp, bnn = accs
        rot = lax.bitwise_and(lane + d, _TD - 1)
        uv = plsc.load_gather(tu, [rows, hu + rot])
        pv = plsc.load_gather(tp, [rows, hp + rot])
        nv = plsc.load_gather(tn, [rows, hn + rot])
        return (bup + uv * pv, bun + uv * nv,
                buu + uv * uv, bpp + pv * pv, bnn + nv * nv)

      bup, bun, buu, bpp, bnn = lax.fori_loop(
          0, _TD, tbody, (z, z, z, z, z), unroll=4)

      wuv = wus[pl.ds(g * 16, 16)]
      wpv = wps[pl.ds(g * 16, 16)]
      wnv = wns[pl.ds(g * 16, 16)]

      o = r * _C + g * 16
      for j, val in enumerate((aup, aun, auu, app, ann,
                               bup, bun, buu, bpp, bnn,
                               wuv, wpv, wnv)):
        staging[pl.ds(j * _PER_W + o, 16)] = val

  for j in range(13):
    pltpu.sync_copy(staging.at[pl.ds(j * _PER_W, _PER_W)],
                    outs[j].at[pl.ds(base, _PER_W)])


_sc_stats = functools.partial(
    pl.kernel,
    out_type=tuple(jax.ShapeDtypeStruct((_B,), jnp.float32)
                   for _ in range(13)),
    mesh=plsc.VectorSubcoreMesh(
        core_axis_name="c", subcore_axis_name="s",
        num_cores=_NC, num_subcores=_NS),
    scratch_types=[
        pltpu.VMEM((_C,), jnp.int32),
        pltpu.VMEM((_C,), jnp.int32),
        pltpu.VMEM((_C,), jnp.int32),
        pltpu.VMEM((_C,), jnp.int32),
        pltpu.VMEM((_C,), jnp.int32),
        pltpu.VMEM((_C,), jnp.int32),
        pltpu.VMEM((_C, _KD), jnp.float32),
        pltpu.VMEM((_C, _KD), jnp.float32),
        pltpu.VMEM((_C, _KD), jnp.float32),
        pltpu.VMEM((_C, _KD), jnp.float32),
        pltpu.VMEM((_C, _KD), jnp.float32),
        pltpu.VMEM((_C, _KD), jnp.float32),
        pltpu.VMEM((_C,), jnp.float32),
        pltpu.VMEM((_C,), jnp.float32),
        pltpu.VMEM((_C,), jnp.float32),
        pltpu.VMEM((13 * _PER_W,), jnp.float32),
        pltpu.SemaphoreType.DMA,
    ],
    compiler_params=pltpu.CompilerParams(needs_layout_passes=False),
)(_sc_body)


def _ep_body(up_r, un_r, uu_r, pp_r, nn_r, tup_r, tun_r, tuu_r, tpp_r,
             tnn_r, wu_r, wp_r, wn_r, o_ref):
  up, un, uu, pp, nn = up_r[...], un_r[...], uu_r[...], pp_r[...], nn_r[...]
  tup, tun, tuu, tpp, tnn = (tup_r[...], tun_r[...], tuu_r[...],
                             tpp_r[...], tnn_r[...])
  wu, wp, wn = wu_r[...], wp_r[...], wn_r[...]

  eps = jnp.float32(1e-12)
  nu = jnp.maximum(jnp.sqrt(tuu), eps)
  np_ = jnp.maximum(jnp.sqrt(tpp), eps)
  nn_ = jnp.maximum(jnp.sqrt(tnn), eps)

  pos = up + wu * wp * tup / (nu * np_)
  neg = un + wu * wn * tun / (nu * nn_)
  l2 = (uu + pp + nn
        + (wu * wu) * tuu / (nu * nu)
        + (wp * wp) * tpp / (np_ * np_)
        + (wn * wn) * tnn / (nn_ * nn_))

  d = pos - neg
  base = jnp.maximum(-d, 0.0) + jnp.log1p(jnp.exp(-jnp.abs(d)))
  loss = jnp.mean(base) + _REG * (jnp.sum(l2) / (2.0 * _B))
  o_ref[0, 0] = loss


def kernel(u, pos_i, neg_i, ua_embed, ea_embed, u_text, i_text,
           w_utext, w_ctext):
  ut2 = jnp.reshape(u_text, (u_text.shape[0] // 2, 2 * _TD))
  it2 = jnp.reshape(i_text, (i_text.shape[0] // 2, 2 * _TD))
  wu1 = jnp.reshape(w_utext, (-1,))
  wc1 = jnp.reshape(w_ctext, (-1,))
  stats = _sc_stats(u, pos_i, neg_i, ua_embed, ea_embed, ut2, it2,
                    wu1, wc1)
  loss = pl.pallas_call(
      _ep_body,
      out_shape=jax.ShapeDtypeStruct((1, 1), jnp.float32),
      out_specs=pl.BlockSpec(memory_space=pltpu.SMEM),
  )(*stats)
  return loss[0, 0]


# speedup vs baseline: 2.9009x; 1.1024x over previous
import functools

import jax
import jax.numpy as jnp
from jax import lax
from jax.experimental import pallas as pl
from jax.experimental.pallas import tpu as pltpu
from jax.experimental.pallas import tpu_sc as plsc

_B = 16384
_KD = 128
_TD = 64
_NC = 2
_NS = 16
_NW = _NC * _NS
_PER_W = _B // _NW
_C = 128
_ROUNDS = _PER_W // _C
_GROUPS = _C // 16
_REG = 1e-05

_MESH = plsc.VectorSubcoreMesh(
    core_axis_name="c", subcore_axis_name="s",
    num_cores=_NC, num_subcores=_NS)
_PARAMS = pltpu.CompilerParams(needs_layout_passes=False)


def _kgat_body(u_ref, pos_ref, neg_ref, ua_ref, ea_ref,
               o0, o1, o2, o3, o4,
               idx_u, idx_p, idx_n, ru, rp, rn, staging, sem):
  outs = (o0, o1, o2, o3, o4)
  wid = lax.axis_index("s") * _NC + lax.axis_index("c")
  base = wid * _PER_W
  lane = lax.iota(jnp.int32, 16)

  for r in range(_ROUNDS):
    off = base + r * _C
    pltpu.sync_copy(u_ref.at[pl.ds(off, _C)], idx_u)
    pltpu.sync_copy(pos_ref.at[pl.ds(off, _C)], idx_p)
    pltpu.sync_copy(neg_ref.at[pl.ds(off, _C)], idx_n)
    cps = [
        pltpu.async_copy(ua_ref.at[idx_u], ru, sem),
        pltpu.async_copy(ea_ref.at[idx_p], rp, sem),
        pltpu.async_copy(ea_ref.at[idx_n], rn, sem),
    ]
    for cp in cps:
      cp.wait()

    for g in range(_GROUPS):
      rows = lane + (g * 16)
      z = jnp.zeros((16,), jnp.float32)

      def kbody(d, accs, rows=rows):
        aup, aun, auu, app, ann = accs
        col = lax.bitwise_and(lane + d, _KD - 1)
        uv = plsc.load_gather(ru, [rows, col])
        pv = plsc.load_gather(rp, [rows, col])
        nv = plsc.load_gather(rn, [rows, col])
        return (aup + uv * pv, aun + uv * nv,
                auu + uv * uv, app + pv * pv, ann + nv * nv)

      accs = lax.fori_loop(0, _KD, kbody, (z, z, z, z, z), unroll=4)

      o = r * _C + g * 16
      for j, val in enumerate(accs):
        staging[pl.ds(j * _PER_W + o, 16)] = val

  for j in range(5):
    pltpu.sync_copy(staging.at[pl.ds(j * _PER_W, _PER_W)],
                    outs[j].at[pl.ds(base, _PER_W)])


_kgat_stats = functools.partial(
    pl.kernel,
    out_type=tuple(jax.ShapeDtypeStruct((_B,), jnp.float32)
                   for _ in range(5)),
    mesh=_MESH,
    scratch_types=[
        pltpu.VMEM((_C,), jnp.int32),
        pltpu.VMEM((_C,), jnp.int32),
        pltpu.VMEM((_C,), jnp.int32),
        pltpu.VMEM((_C, _KD), jnp.float32),
        pltpu.VMEM((_C, _KD), jnp.float32),
        pltpu.VMEM((_C, _KD), jnp.float32),
        pltpu.VMEM((5 * _PER_W,), jnp.float32),
        pltpu.SemaphoreType.DMA,
    ],
    compiler_params=_PARAMS,
)(_kgat_body)


def _text_body(u_ref, pos_ref, neg_ref, ut_ref, it_ref, wu_ref, wc_ref,
               o0, o1, o2, o3, o4, o5, o6, o7,
               idx_u, idx_p, idx_n, tix_u, tix_p, tix_n,
               tu, tp, tn, wus, wps, wns, staging, sem):
  outs = (o0, o1, o2, o3, o4, o5, o6, o7)
  wid = lax.axis_index("s") * _NC + lax.axis_index("c")
  base = wid * _PER_W
  lane = lax.iota(jnp.int32, 16)

  for r in range(_ROUNDS):
    off = base + r * _C
    pltpu.sync_copy(u_ref.at[pl.ds(off, _C)], idx_u)
    pltpu.sync_copy(pos_ref.at[pl.ds(off, _C)], idx_p)
    pltpu.sync_copy(neg_ref.at[pl.ds(off, _C)], idx_n)
    for g in range(_GROUPS):
      sl = pl.ds(g * 16, 16)
      tix_u[sl] = lax.shift_right_logical(idx_u[sl], 1)
      tix_p[sl] = lax.shift_right_logical(idx_p[sl], 1)
      tix_n[sl] = lax.shift_right_logical(idx_n[sl], 1)
    cps = [
        pltpu.async_copy(ut_ref.at[tix_u], tu, sem),
        pltpu.async_copy(it_ref.at[tix_p], tp, sem),
        pltpu.async_copy(it_ref.at[tix_n], tn, sem),
        pltpu.async_copy(wu_ref.at[idx_u], wus, sem),
        pltpu.async_copy(wc_ref.at[idx_p], wps, sem),
        pltpu.async_copy(wc_ref.at[idx_n], wns, sem),
    ]
    for cp in cps:
      cp.wait()

    for g in range(_GROUPS):
      rows = lane + (g * 16)
      z = jnp.zeros((16,), jnp.float32)

      half_u = lax.bitwise_and(idx_u[pl.ds(g * 16, 16)], 1) * _TD
      half_p = lax.bitwise_and(idx_p[pl.ds(g * 16, 16)], 1) * _TD
      half_n = lax.bitwise_and(idx_n[pl.ds(g * 16, 16)], 1) * _TD

      def tbody(d, accs, rows=rows, hu=half_u, hp=half_p, hn=half_n):
        bup, bun, buu, bpp, bnn = accs
        rot = lax.bitwise_and(lane + d, _TD - 1)
        uv = plsc.load_gather(tu, [rows, hu + rot])
        pv = plsc.load_gather(tp, [rows, hp + rot])
        nv = plsc.load_gather(tn, [rows, hn + rot])
        return (bup + uv * pv, bun + uv * nv,
                buu + uv * uv, bpp + pv * pv, bnn + nv * nv)

      accs = lax.fori_loop(0, _TD, tbody, (z, z, z, z, z), unroll=4)

      wuv = wus[pl.ds(g * 16, 16)]
      wpv = wps[pl.ds(g * 16, 16)]
      wnv = wns[pl.ds(g * 16, 16)]

      o = r * _C + g * 16
      for j, val in enumerate(accs + (wuv, wpv, wnv)):
        staging[pl.ds(j * _PER_W + o, 16)] = val

  for j in range(8):
    pltpu.sync_copy(staging.at[pl.ds(j * _PER_W, _PER_W)],
                    outs[j].at[pl.ds(base, _PER_W)])


_text_stats = functools.partial(
    pl.kernel,
    out_type=tuple(jax.ShapeDtypeStruct((_B,), jnp.float32)
                   for _ in range(8)),
    mesh=_MESH,
    scratch_types=[
        pltpu.VMEM((_C,), jnp.int32),
        pltpu.VMEM((_C,), jnp.int32),
        pltpu.VMEM((_C,), jnp.int32),
        pltpu.VMEM((_C,), jnp.int32),
        pltpu.VMEM((_C,), jnp.int32),
        pltpu.VMEM((_C,), jnp.int32),
        pltpu.VMEM((_C, _KD), jnp.float32),
        pltpu.VMEM((_C, _KD), jnp.float32),
        pltpu.VMEM((_C, _KD), jnp.float32),
        pltpu.VMEM((_C,), jnp.float32),
        pltpu.VMEM((_C,), jnp.float32),
        pltpu.VMEM((_C,), jnp.float32),
        pltpu.VMEM((8 * _PER_W,), jnp.float32),
        pltpu.SemaphoreType.DMA,
    ],
    compiler_params=_PARAMS,
)(_text_body)


def _ep_body(up_r, un_r, uu_r, pp_r, nn_r, tup_r, tun_r, tuu_r, tpp_r,
             tnn_r, wu_r, wp_r, wn_r, o_ref):
  up, un, uu, pp, nn = up_r[...], un_r[...], uu_r[...], pp_r[...], nn_r[...]
  tup, tun, tuu, tpp, tnn = (tup_r[...], tun_r[...], tuu_r[...],
                             tpp_r[...], tnn_r[...])
  wu, wp, wn = wu_r[...], wp_r[...], wn_r[...]

  eps = jnp.float32(1e-12)
  nu = jnp.maximum(jnp.sqrt(tuu), eps)
  np_ = jnp.maximum(jnp.sqrt(tpp), eps)
  nn_ = jnp.maximum(jnp.sqrt(tnn), eps)

  pos = up + wu * wp * tup / (nu * np_)
  neg = un + wu * wn * tun / (nu * nn_)
  l2 = (uu + pp + nn
        + (wu * wu) * tuu / (nu * nu)
        + (wp * wp) * tpp / (np_ * np_)
        + (wn * wn) * tnn / (nn_ * nn_))

  d = pos - neg
  base = jnp.maximum(-d, 0.0) + jnp.log1p(jnp.exp(-jnp.abs(d)))
  loss = jnp.mean(base) + _REG * (jnp.sum(l2) / (2.0 * _B))
  o_ref[0, 0] = loss


def kernel(u, pos_i, neg_i, ua_embed, ea_embed, u_text, i_text,
           w_utext, w_ctext):
  ut2 = jnp.reshape(u_text, (u_text.shape[0] // 2, 2 * _TD))
  it2 = jnp.reshape(i_text, (i_text.shape[0] // 2, 2 * _TD))
  wu1 = w_utext[:, 0]
  wc1 = w_ctext[:, 0]
  kstats = _kgat_stats(u, pos_i, neg_i, ua_embed, ea_embed)
  tstats = _text_stats(u, pos_i, neg_i, ut2, it2, wu1, wc1)
  loss = pl.pallas_call(
      _ep_body,
      out_shape=jax.ShapeDtypeStruct((1, 1), jnp.float32),
      out_specs=pl.BlockSpec(memory_space=pltpu.SMEM),
  )(*(kstats + tstats))
  return loss[0, 0]
